# baseline (speedup 1.0000x reference)
import functools

import jax
import jax.numpy as jnp
from jax import lax
from jax.experimental import pallas as pl
from jax.experimental.pallas import tpu as pltpu
from jax.experimental.pallas import tpu_sc as plsc

N_NODES = 10000
N_PAD = 10240
NPW = 320
E_REAL = 170000
E_PAD = 170240
C = 64
NC, NS = 2, 16
NW = NC * NS
L16 = 16


def _proj_body(relu, s_ref, wl_ref, bl_ref, wr_ref, br_ref, xl_ref, xr_ref):
    s = s_ref[...]
    if relu:
        s = jnp.maximum(s, 0.0)
    xl_ref[...] = jnp.dot(s, wl_ref[...], preferred_element_type=jnp.float32) + bl_ref[...]
    xr_ref[...] = jnp.dot(s, wr_ref[...], preferred_element_type=jnp.float32) + br_ref[...]


def _proj(state, Wl, bl, Wr, br, relu):
    n, din = state.shape
    dout = Wl.shape[1]
    p = max(dout, 128)
    if p != dout:
        Wl = jnp.pad(Wl, ((0, 0), (0, p - dout)))
        Wr = jnp.pad(Wr, ((0, 0), (0, p - dout)))
        bl = jnp.pad(bl, (0, p - dout))
        br = jnp.pad(br, (0, p - dout))
    rb = 640
    return pl.pallas_call(
        functools.partial(_proj_body, relu),
        grid=(n // rb,),
        in_specs=[
            pl.BlockSpec((rb, din), lambda i: (i, 0)),
            pl.BlockSpec((din, p), lambda i: (0, 0)),
            pl.BlockSpec((1, p), lambda i: (0, 0)),
            pl.BlockSpec((din, p), lambda i: (0, 0)),
            pl.BlockSpec((1, p), lambda i: (0, 0)),
        ],
        out_specs=[
            pl.BlockSpec((rb, p), lambda i: (i, 0)),
            pl.BlockSpec((rb, p), lambda i: (i, 0)),
        ],
        out_shape=[
            jax.ShapeDtypeStruct((n, p), jnp.float32),
            jax.ShapeDtypeStruct((n, p), jnp.float32),
        ],
    )(state, Wl, bl.reshape(1, p), Wr, br.reshape(1, p))


def _ew_body(ea_ref, w_ref, o_ref):
    o_ref[...] = jnp.dot(ea_ref[...], w_ref[...], preferred_element_type=jnp.float32)


def _ew_matmul(ea, We):
    e, k = ea.shape
    dout = We.shape[1]
    rb = 1280
    return pl.pallas_call(
        _ew_body,
        grid=(e // rb,),
        in_specs=[
            pl.BlockSpec((rb, k), lambda i: (i, 0)),
            pl.BlockSpec((k, dout), lambda i: (0, 0)),
        ],
        out_specs=pl.BlockSpec((rb, dout), lambda i: (i, 0)),
        out_shape=jax.ShapeDtypeStruct((e, dout), jnp.float32),
    )(ea, We)


_DNUMS = lax.GatherDimensionNumbers(
    offset_dims=(), collapsed_slice_dims=(0,), start_index_map=(0,))


def _allsum(v):
    lanes = lax.broadcasted_iota(jnp.int32, (L16,), 0)
    for sh in (8, 4, 2, 1):
        idx = (lanes ^ sh).reshape(L16, 1)
        v = v + lax.gather(v, idx, _DNUMS, (1,),
                           mode=lax.GatherScatterMode.PROMISE_IN_BOUNDS)
    return v


def _recip(x):
    xi = lax.bitcast_convert_type(x, jnp.int32)
    r = lax.bitcast_convert_type(jnp.int32(0x7EF311C3) - xi, jnp.float32)
    for _ in range(3):
        r = r * (2.0 - x * r)
    return r


def _sc_edge_stage(dout):
    G = dout // L16
    P = max(dout, 128)
    CD = 128 if dout <= 128 else 64
    csh = 7 if dout <= 128 else 6
    mesh = plsc.VectorSubcoreMesh(core_axis_name="c", subcore_axis_name="s")

    @functools.partial(
        pl.kernel,
        out_type=jax.ShapeDtypeStruct((N_NODES, dout), jnp.float32),
        mesh=mesh,
        scratch_types=[
            pltpu.VMEM((64,), jnp.int32),
            pltpu.VMEM((CD + L16,), jnp.int32),
            pltpu.VMEM((CD + L16,), jnp.int32),
            pltpu.VMEM((CD + L16,), jnp.int32),
            pltpu.VMEM((CD + L16,), jnp.int32),
            pltpu.VMEM((CD, P), jnp.float32),
            pltpu.VMEM((CD, P), jnp.float32),
            pltpu.VMEM((CD, P), jnp.float32),
            pltpu.VMEM((CD, P), jnp.float32),
            pltpu.VMEM((CD, dout), jnp.float32),
            pltpu.VMEM((CD, dout), jnp.float32),
            pltpu.VMEM((dout,), jnp.float32),
            pltpu.VMEM((dout,), jnp.float32),
            pltpu.VMEM((dout,), jnp.float32),
            pltpu.VMEM((dout,), jnp.float32),
            pltpu.VMEM((dout,), jnp.float32),
            pltpu.SemaphoreType.DMA,
            pltpu.SemaphoreType.DMA,
            pltpu.SemaphoreType.DMA,
            pltpu.SemaphoreType.DMA,
            pltpu.SemaphoreType.DMA,
            pltpu.SemaphoreType.DMA,
            pltpu.SemaphoreType.DMA,
            pltpu.SemaphoreType.DMA,
            pltpu.SemaphoreType.DMA,
            pltpu.SemaphoreType.DMA,
        ],
    )
    def k(xl_hbm, xr_hbm, ew_hbm, srcc_hbm, dstc_hbm, offs_hbm, att_hbm,
          bias_hbm, out_hbm, offs_v, src_a, src_b, dst_a, dst_b,
          xl_a, xl_b, xr_a, xr_b, ew_a, ew_b, u_v, orow_a, orow_b,
          att_v, bias_v,
          sem_si, sem_di, sem_ewa, sem_ewb, sem_xla, sem_xlb,
          sem_xra, sem_xrb, sem_sta, sem_stb):
        wid = lax.axis_index("s") * NC + lax.axis_index("c")

        pltpu.sync_copy(offs_hbm, offs_v.at[pl.ds(0, 48)])
        pltpu.sync_copy(att_hbm, att_v)
        pltpu.sync_copy(bias_hbm, bias_v)
        z16 = jnp.zeros((L16,), jnp.float32)
        for g in range(G):
            u_v[pl.ds(g * L16, L16)] = z16
        def zrow(r, dummy):
            for buf in (xl_a, xl_b, xr_a, xr_b):
                for g in range(P // L16):
                    buf[r, pl.ds(g * L16, L16)] = z16
            for buf in (ew_a, ew_b):
                for g in range(G):
                    buf[r, pl.ds(g * L16, L16)] = z16
            return dummy

        lax.fori_loop(0, CD, zrow, jnp.int32(0))
        att_regs = [att_v[pl.ds(g * L16, L16)] for g in range(G)]

        neg = jnp.float32(-1e30)

        e0 = offs_v[pl.ds(wid, L16)][0]
        e1 = offs_v[pl.ds(wid + 1, L16)][0]
        c_lo = lax.shift_right_logical(e0, csh)
        c_hi = lax.shift_right_logical(e1 + (CD - 1), csh)
        nchunks = c_hi - c_lo

        slot_a = (src_a, dst_a, xl_a, xr_a, ew_a, sem_ewa, sem_xla, sem_xra)
        slot_b = (src_b, dst_b, xl_b, xr_b, ew_b, sem_ewb, sem_xlb, sem_xrb)

        def issue(ci, sl):
            srcv, dstv, xlv, xrv, ewv, s_ew, s_xl, s_xr = sl
            h1 = pltpu.async_copy(srcc_hbm.at[pl.ds(ci * CD, CD)],
                                  srcv.at[pl.ds(0, CD)], sem_si)
            h2 = pltpu.async_copy(dstc_hbm.at[pl.ds(ci * CD, CD)],
                                  dstv.at[pl.ds(0, CD)], sem_di)
            h1.wait()
            h2.wait()
            pltpu.async_copy(ew_hbm.at[pl.ds(ci * CD, CD)], ewv, s_ew)
            pltpu.async_copy(xl_hbm.at[srcv.at[pl.ds(0, CD)]], xlv, s_xl)
            pltpu.async_copy(xr_hbm.at[dstv.at[pl.ds(0, CD)]], xrv, s_xr)

        def wait_chunk(ci, sl):
            srcv, dstv, xlv, xrv, ewv, s_ew, s_xl, s_xr = sl
            pltpu.make_async_copy(ew_hbm.at[pl.ds(ci * CD, CD)], ewv,
                                  s_ew).wait()
            pltpu.make_async_copy(xl_hbm.at[srcv.at[pl.ds(0, CD)]], xlv,
                                  s_xl).wait()
            pltpu.make_async_copy(xr_hbm.at[dstv.at[pl.ds(0, CD)]], xrv,
                                  s_xr).wait()

        def finalize(nd_prev, dv, seg_cnt):
            rec = _recip(dv + 1e-16)
            par = jnp.bitwise_and(seg_cnt, 1)

            @pl.when(par == 0)
            def _():
                @pl.when(seg_cnt >= 2)
                def _():
                    pltpu.make_async_copy(orow_a, out_hbm.at[nd_prev],
                                          sem_sta).wait()
                for gg in range(G):
                    slc = pl.ds(gg * L16, L16)
                    orow_a[slc] = u_v[slc] * rec + bias_v[slc]
                pltpu.async_copy(orow_a, out_hbm.at[nd_prev], sem_sta)

            @pl.when(par == 1)
            def _():
                @pl.when(seg_cnt >= 3)
                def _():
                    pltpu.make_async_copy(orow_b, out_hbm.at[nd_prev],
                                          sem_stb).wait()
                for gg in range(G):
                    slc = pl.ds(gg * L16, L16)
                    orow_b[slc] = u_v[slc] * rec + bias_v[slc]
                pltpu.async_copy(orow_b, out_hbm.at[nd_prev], sem_stb)

        def edge_loop(ci, sl, carry):
            dstv, xlv, xrv, ewv = sl[1], sl[2], sl[3], sl[4]

            def edge_body(j, ecarry):
                nd_prev, mv, dv, seg_cnt = ecarry
                e_glob = ci * CD + j
                active = jnp.logical_and(e_glob >= e0, e_glob < e1)
                nd = dstv[pl.ds(j, L16)][0]
                newseg = jnp.logical_and(active, nd != nd_prev)
                fin = jnp.logical_and(newseg, nd_prev >= 0)

                @pl.when(fin)
                def _():
                    finalize(nd_prev, dv, seg_cnt)

                seg_cnt = seg_cnt + fin.astype(jnp.int32)

                nsv = jnp.full((L16,), newseg.astype(jnp.float32))
                mv = mv + nsv * (neg - mv)

                acc = jnp.zeros((L16,), jnp.float32)
                for gg in range(G):
                    slc = pl.ds(gg * L16, L16)
                    u = xlv[j, slc] + xrv[j, slc] + ewv[j, slc]
                    acc = acc + att_regs[gg] * jnp.maximum(u, 0.2 * u)
                lv = _allsum(acc)
                afv = jnp.full((L16,), active.astype(jnp.float32))
                lv_m = lv + (1.0 - afv) * (neg - lv)
                m_new = jnp.maximum(mv, lv_m)
                cf = jnp.exp(mv - m_new)
                wv = jnp.exp(lv_m - m_new)
                dv = dv * cf + wv
                for gg in range(G):
                    slc = pl.ds(gg * L16, L16)
                    u_v[slc] = u_v[slc] * cf + wv * xlv[j, slc]
                nd_prev = jnp.where(active, nd, nd_prev)
                return nd_prev, m_new, dv, seg_cnt

            return lax.fori_loop(0, CD, edge_body, carry)

        @pl.when(nchunks > 0)
        def _():
            issue(c_lo, slot_a)

        def pair_body(i2, carry):
            ci = c_lo + 2 * i2

            @pl.when(ci + 1 < c_hi)
            def _():
                issue(ci + 1, slot_b)

            wait_chunk(ci, slot_a)
            carry = edge_loop(ci, slot_a, carry)

            @pl.when(ci + 2 < c_hi)
            def _():
                issue(ci + 2, slot_a)

            @pl.when(ci + 1 < c_hi)
            def _():
                wait_chunk(ci + 1, slot_b)

            carry = edge_loop(ci + 1, slot_b, carry)
            return carry

        init = (jnp.int32(-1), jnp.full((L16,), neg),
                jnp.zeros((L16,), jnp.float32), jnp.int32(0))
        npairs = lax.shift_right_logical(nchunks + 1, 1)
        nd_prev, mv, dv, seg_cnt = lax.fori_loop(0, npairs, pair_body, init)

        @pl.when(nd_prev >= 0)
        def _():
            finalize(nd_prev, dv, seg_cnt)

        total = seg_cnt + (nd_prev >= 0).astype(jnp.int32)

        @pl.when(total >= 1)
        def _():
            pltpu.make_async_copy(orow_a, out_hbm.at[0], sem_sta).wait()

        @pl.when(total >= 2)
        def _():
            pltpu.make_async_copy(orow_b, out_hbm.at[0], sem_stb).wait()

    return k


_SC_KERNELS = {d: _sc_edge_stage(d) for d in (32, 64, 128, 256)}


def kernel(x, edge_index, edge_attr, batch, params, n_steps):
    n = x.shape[0]
    src = edge_index[0]
    dst = edge_index[1]
    loop = jnp.arange(n, dtype=src.dtype)
    src_all = jnp.concatenate([src, loop])
    dst_all = jnp.concatenate([dst, loop])
    perm = jnp.argsort(dst_all)
    src_s = src_all[perm]
    dst_s = dst_all[perm]
    loop_attr = jnp.broadcast_to(jnp.mean(edge_attr, axis=0), (n, edge_attr.shape[1]))
    ea_all = jnp.concatenate([edge_attr, loop_attr], axis=0)
    ea_s = ea_all[perm]

    npad = E_PAD - E_REAL
    src_sp = jnp.concatenate([src_s, jnp.full((npad,), N_NODES, jnp.int32)])
    dst_sp = jnp.concatenate([dst_s, jnp.full((npad,), N_NODES + 1, jnp.int32)])
    ea_sp = jnp.concatenate(
        [ea_s, jnp.zeros((npad, ea_s.shape[1]), jnp.float32)], axis=0)

    bnds = jnp.arange(33, dtype=jnp.int32) * NPW
    offs33 = jnp.searchsorted(dst_s, bnds, side="left").astype(jnp.int32)
    offs48 = jnp.concatenate([offs33, jnp.full((15,), E_REAL, jnp.int32)])

    state = jnp.pad(x, ((0, N_PAD - n), (0, 0)))
    out = None
    for i, p in enumerate(params):
        dout = p["Wl"].shape[1]
        xl, xr = _proj(state, p["Wl"], p["bl"], p["Wr"], p["br"], relu=(i > 0))
        ew = _ew_matmul(ea_sp, p["We"])
        out = _SC_KERNELS[dout](xl, xr, ew, src_sp, dst_sp, offs48,
                                p["att"], p["bias"])
        if i < len(params) - 1:
            state = jnp.pad(out, ((0, N_PAD - n), (0, 0)))
    return out


# speedup vs baseline: 1.0547x; 1.0547x over previous
import functools

import jax
import jax.numpy as jnp
from jax import lax
from jax.experimental import pallas as pl
from jax.experimental.pallas import tpu as pltpu
from jax.experimental.pallas import tpu_sc as plsc

N_NODES = 10000
N_PAD = 10240
NPW = 320
E_REAL = 170000
E_PAD = 170240
C = 64
NC, NS = 2, 16
NW = NC * NS
L16 = 16


def _proj_body(relu, s_ref, wl_ref, bl_ref, wr_ref, br_ref, xl_ref, xr_ref):
    s = s_ref[...]
    if relu:
        s = jnp.maximum(s, 0.0)
    xl_ref[...] = jnp.dot(s, wl_ref[...], preferred_element_type=jnp.float32) + bl_ref[...]
    xr_ref[...] = jnp.dot(s, wr_ref[...], preferred_element_type=jnp.float32) + br_ref[...]


def _proj(state, Wl, bl, Wr, br, relu):
    n, din = state.shape
    dout = Wl.shape[1]
    p = max(dout, 128)
    if p != dout:
        Wl = jnp.pad(Wl, ((0, 0), (0, p - dout)))
        Wr = jnp.pad(Wr, ((0, 0), (0, p - dout)))
        bl = jnp.pad(bl, (0, p - dout))
        br = jnp.pad(br, (0, p - dout))
    rb = 640
    return pl.pallas_call(
        functools.partial(_proj_body, relu),
        grid=(n // rb,),
        in_specs=[
            pl.BlockSpec((rb, din), lambda i: (i, 0)),
            pl.BlockSpec((din, p), lambda i: (0, 0)),
            pl.BlockSpec((1, p), lambda i: (0, 0)),
            pl.BlockSpec((din, p), lambda i: (0, 0)),
            pl.BlockSpec((1, p), lambda i: (0, 0)),
        ],
        out_specs=[
            pl.BlockSpec((rb, p), lambda i: (i, 0)),
            pl.BlockSpec((rb, p), lambda i: (i, 0)),
        ],
        out_shape=[
            jax.ShapeDtypeStruct((n, p), jnp.float32),
            jax.ShapeDtypeStruct((n, p), jnp.float32),
        ],
    )(state, Wl, bl.reshape(1, p), Wr, br.reshape(1, p))


def _ew_body(ea_ref, w_ref, o_ref):
    o_ref[...] = jnp.dot(ea_ref[...], w_ref[...], preferred_element_type=jnp.float32)


def _ew_matmul(ea, We):
    e, k = ea.shape
    dout = We.shape[1]
    rb = 1280
    return pl.pallas_call(
        _ew_body,
        grid=(e // rb,),
        in_specs=[
            pl.BlockSpec((rb, k), lambda i: (i, 0)),
            pl.BlockSpec((k, dout), lambda i: (0, 0)),
        ],
        out_specs=pl.BlockSpec((rb, dout), lambda i: (i, 0)),
        out_shape=jax.ShapeDtypeStruct((e, dout), jnp.float32),
    )(ea, We)


_DNUMS = lax.GatherDimensionNumbers(
    offset_dims=(), collapsed_slice_dims=(0,), start_index_map=(0,))


def _allsum(v):
    lanes = lax.broadcasted_iota(jnp.int32, (L16,), 0)
    for sh in (8, 4, 2, 1):
        idx = (lanes ^ sh).reshape(L16, 1)
        v = v + lax.gather(v, idx, _DNUMS, (1,),
                           mode=lax.GatherScatterMode.PROMISE_IN_BOUNDS)
    return v


def _recip(x):
    xi = lax.bitcast_convert_type(x, jnp.int32)
    r = lax.bitcast_convert_type(jnp.int32(0x7EF311C3) - xi, jnp.float32)
    for _ in range(3):
        r = r * (2.0 - x * r)
    return r


def _sc_edge_stage(dout):
    G = dout // L16
    P = max(dout, 128)
    CD = 128 if dout <= 128 else 64
    csh = 7 if dout <= 128 else 6
    mesh = plsc.VectorSubcoreMesh(core_axis_name="c", subcore_axis_name="s")

    @functools.partial(
        pl.kernel,
        out_type=jax.ShapeDtypeStruct((N_NODES, dout), jnp.float32),
        mesh=mesh,
        scratch_types=[
            pltpu.VMEM((64,), jnp.int32),
            pltpu.VMEM((CD + L16,), jnp.int32),
            pltpu.VMEM((CD + L16,), jnp.int32),
            pltpu.VMEM((CD + L16,), jnp.int32),
            pltpu.VMEM((CD + L16,), jnp.int32),
            pltpu.VMEM((CD, P), jnp.float32),
            pltpu.VMEM((CD, P), jnp.float32),
            pltpu.VMEM((CD, P), jnp.float32),
            pltpu.VMEM((CD, P), jnp.float32),
            pltpu.VMEM((CD, dout), jnp.float32),
            pltpu.VMEM((CD, dout), jnp.float32),
            pltpu.VMEM((dout,), jnp.float32),
            pltpu.VMEM((dout,), jnp.float32),
            pltpu.VMEM((dout,), jnp.float32),
            pltpu.VMEM((dout,), jnp.float32),
            pltpu.VMEM((dout,), jnp.float32),
            pltpu.SemaphoreType.DMA,
            pltpu.SemaphoreType.DMA,
            pltpu.SemaphoreType.DMA,
            pltpu.SemaphoreType.DMA,
            pltpu.SemaphoreType.DMA,
            pltpu.SemaphoreType.DMA,
            pltpu.SemaphoreType.DMA,
            pltpu.SemaphoreType.DMA,
            pltpu.SemaphoreType.DMA,
            pltpu.SemaphoreType.DMA,
        ],
    )
    def k(xl_hbm, xr_hbm, ew_hbm, srcc_hbm, dstc_hbm, offs_hbm, att_hbm,
          bias_hbm, out_hbm, offs_v, src_a, src_b, dst_a, dst_b,
          xl_a, xl_b, xr_a, xr_b, ew_a, ew_b, u_v, orow_a, orow_b,
          att_v, bias_v,
          sem_si, sem_di, sem_ewa, sem_ewb, sem_xla, sem_xlb,
          sem_xra, sem_xrb, sem_sta, sem_stb):
        wid = lax.axis_index("s") * NC + lax.axis_index("c")

        pltpu.sync_copy(offs_hbm, offs_v.at[pl.ds(0, 48)])
        pltpu.sync_copy(att_hbm, att_v)
        pltpu.sync_copy(bias_hbm, bias_v)
        z16 = jnp.zeros((L16,), jnp.float32)
        for g in range(G):
            u_v[pl.ds(g * L16, L16)] = z16
        att_regs = [att_v[pl.ds(g * L16, L16)] for g in range(G)]

        neg = jnp.float32(-1e30)

        e0 = offs_v[pl.ds(wid, L16)][0]
        e1 = offs_v[pl.ds(wid + 1, L16)][0]
        c_lo = lax.shift_right_logical(e0, csh)
        c_hi = lax.shift_right_logical(e1 + (CD - 1), csh)
        nchunks = c_hi - c_lo

        slot_a = (src_a, dst_a, xl_a, xr_a, ew_a, sem_ewa, sem_xla, sem_xra)
        slot_b = (src_b, dst_b, xl_b, xr_b, ew_b, sem_ewb, sem_xlb, sem_xrb)

        def issue(ci, sl):
            srcv, dstv, xlv, xrv, ewv, s_ew, s_xl, s_xr = sl
            h1 = pltpu.async_copy(srcc_hbm.at[pl.ds(ci * CD, CD)],
                                  srcv.at[pl.ds(0, CD)], sem_si)
            h2 = pltpu.async_copy(dstc_hbm.at[pl.ds(ci * CD, CD)],
                                  dstv.at[pl.ds(0, CD)], sem_di)
            h1.wait()
            h2.wait()
            pltpu.async_copy(ew_hbm.at[pl.ds(ci * CD, CD)], ewv, s_ew)
            pltpu.async_copy(xl_hbm.at[srcv.at[pl.ds(0, CD)]], xlv, s_xl)
            pltpu.async_copy(xr_hbm.at[dstv.at[pl.ds(0, CD)]], xrv, s_xr)

        def wait_chunk(ci, sl):
            srcv, dstv, xlv, xrv, ewv, s_ew, s_xl, s_xr = sl
            pltpu.make_async_copy(ew_hbm.at[pl.ds(ci * CD, CD)], ewv,
                                  s_ew).wait()
            pltpu.make_async_copy(xl_hbm.at[srcv.at[pl.ds(0, CD)]], xlv,
                                  s_xl).wait()
            pltpu.make_async_copy(xr_hbm.at[dstv.at[pl.ds(0, CD)]], xrv,
                                  s_xr).wait()

        def finalize(nd_prev, dv, seg_cnt):
            rec = _recip(dv + 1e-16)
            par = jnp.bitwise_and(seg_cnt, 1)

            @pl.when(par == 0)
            def _():
                @pl.when(seg_cnt >= 2)
                def _():
                    pltpu.make_async_copy(orow_a, out_hbm.at[nd_prev],
                                          sem_sta).wait()
                for gg in range(G):
                    slc = pl.ds(gg * L16, L16)
                    orow_a[slc] = u_v[slc] * rec + bias_v[slc]
                pltpu.async_copy(orow_a, out_hbm.at[nd_prev], sem_sta)

            @pl.when(par == 1)
            def _():
                @pl.when(seg_cnt >= 3)
                def _():
                    pltpu.make_async_copy(orow_b, out_hbm.at[nd_prev],
                                          sem_stb).wait()
                for gg in range(G):
                    slc = pl.ds(gg * L16, L16)
                    orow_b[slc] = u_v[slc] * rec + bias_v[slc]
                pltpu.async_copy(orow_b, out_hbm.at[nd_prev], sem_stb)

        def edge_loop(ci, sl, carry):
            dstv, xlv, xrv, ewv = sl[1], sl[2], sl[3], sl[4]
            base = ci * CD
            j_lo = jnp.maximum(e0 - base, 0)
            j_hi = jnp.maximum(jnp.minimum(e1 - base, CD), j_lo)

            def edge_body(j, ecarry):
                nd_prev, mv, dv, seg_cnt = ecarry
                nd = dstv[pl.ds(j, L16)][0]
                newseg = nd != nd_prev
                fin = jnp.logical_and(newseg, nd_prev >= 0)

                @pl.when(fin)
                def _():
                    finalize(nd_prev, dv, seg_cnt)

                seg_cnt = seg_cnt + fin.astype(jnp.int32)

                nsv = jnp.full((L16,), newseg.astype(jnp.float32))
                mv = mv + nsv * (neg - mv)

                acc = jnp.zeros((L16,), jnp.float32)
                for gg in range(G):
                    slc = pl.ds(gg * L16, L16)
                    u = xlv[j, slc] + xrv[j, slc] + ewv[j, slc]
                    acc = acc + att_regs[gg] * jnp.maximum(u, 0.2 * u)
                lv = _allsum(acc)
                m_new = jnp.maximum(mv, lv)
                cf = jnp.exp(mv - m_new)
                wv = jnp.exp(lv - m_new)
                dv = dv * cf + wv
                for gg in range(G):
                    slc = pl.ds(gg * L16, L16)
                    u_v[slc] = u_v[slc] * cf + wv * xlv[j, slc]
                return nd, m_new, dv, seg_cnt

            return lax.fori_loop(j_lo, j_hi, edge_body, carry)

        @pl.when(nchunks > 0)
        def _():
            issue(c_lo, slot_a)

        def pair_body(i2, carry):
            ci = c_lo + 2 * i2

            @pl.when(ci + 1 < c_hi)
            def _():
                issue(ci + 1, slot_b)

            wait_chunk(ci, slot_a)
            carry = edge_loop(ci, slot_a, carry)

            @pl.when(ci + 2 < c_hi)
            def _():
                issue(ci + 2, slot_a)

            @pl.when(ci + 1 < c_hi)
            def _():
                wait_chunk(ci + 1, slot_b)

            carry = edge_loop(ci + 1, slot_b, carry)
            return carry

        init = (jnp.int32(-1), jnp.full((L16,), neg),
                jnp.zeros((L16,), jnp.float32), jnp.int32(0))
        npairs = lax.shift_right_logical(nchunks + 1, 1)
        nd_prev, mv, dv, seg_cnt = lax.fori_loop(0, npairs, pair_body, init)

        @pl.when(nd_prev >= 0)
        def _():
            finalize(nd_prev, dv, seg_cnt)

        total = seg_cnt + (nd_prev >= 0).astype(jnp.int32)

        @pl.when(total >= 1)
        def _():
            pltpu.make_async_copy(orow_a, out_hbm.at[0], sem_sta).wait()

        @pl.when(total >= 2)
        def _():
            pltpu.make_async_copy(orow_b, out_hbm.at[0], sem_stb).wait()

    return k


_SC_KERNELS = {d: _sc_edge_stage(d) for d in (32, 64, 128, 256)}


def kernel(x, edge_index, edge_attr, batch, params, n_steps):
    n = x.shape[0]
    src = edge_index[0]
    dst = edge_index[1]
    loop = jnp.arange(n, dtype=src.dtype)
    src_all = jnp.concatenate([src, loop])
    dst_all = jnp.concatenate([dst, loop])
    perm = jnp.argsort(dst_all)
    src_s = src_all[perm]
    dst_s = dst_all[perm]
    loop_attr = jnp.broadcast_to(jnp.mean(edge_attr, axis=0), (n, edge_attr.shape[1]))
    ea_all = jnp.concatenate([edge_attr, loop_attr], axis=0)
    ea_s = ea_all[perm]

    npad = E_PAD - E_REAL
    src_sp = jnp.concatenate([src_s, jnp.full((npad,), N_NODES, jnp.int32)])
    dst_sp = jnp.concatenate([dst_s, jnp.full((npad,), N_NODES + 1, jnp.int32)])
    ea_sp = jnp.concatenate(
        [ea_s, jnp.zeros((npad, ea_s.shape[1]), jnp.float32)], axis=0)

    bnds = jnp.arange(33, dtype=jnp.int32) * NPW
    offs33 = jnp.searchsorted(dst_s, bnds, side="left").astype(jnp.int32)
    offs48 = jnp.concatenate([offs33, jnp.full((15,), E_REAL, jnp.int32)])

    state = jnp.pad(x, ((0, N_PAD - n), (0, 0)))
    out = None
    for i, p in enumerate(params):
        dout = p["Wl"].shape[1]
        xl, xr = _proj(state, p["Wl"], p["bl"], p["Wr"], p["br"], relu=(i > 0))
        ew = _ew_matmul(ea_sp, p["We"])
        out = _SC_KERNELS[dout](xl, xr, ew, src_sp, dst_sp, offs48,
                                p["att"], p["bias"])
        if i < len(params) - 1:
            state = jnp.pad(out, ((0, N_PAD - n), (0, 0)))
    return out
